# baseline (speedup 1.0000x reference)
import jax
import jax.numpy as jnp
from jax.experimental import pallas as pl
from jax.experimental.pallas import tpu as pltpu

BB = 8


def _body(x_ref, wz_ref, lzw_ref, bz_ref, lzb_ref, wh_ref, lhw_ref, bh_ref,
          lhb_ref, att_ref, o_ref, adj_ref, gbd_ref, bias_ref, probs_ref):
    i = pl.program_id(0)
    periods = x_ref.shape[0]
    n = x_ref.shape[1]
    f = wz_ref.shape[0]
    hidden = lzw_ref.shape[1]
    zw = BB * hidden

    @pl.when(i == 0)
    def _init_scratch():
        r = jax.lax.broadcasted_iota(jnp.int32, (n, n), 0)
        c = jax.lax.broadcasted_iota(jnp.int32, (n, n), 1)
        adj_ref[...] = jnp.where(
            r >= c,
            jax.lax.rsqrt(r.astype(jnp.float32) + 1.0) *
            jax.lax.rsqrt(c.astype(jnp.float32) + 1.0),
            0.0).astype(jnp.bfloat16)
        lzw1 = lzw_ref[0:hidden, :]
        lhw1 = lhw_ref[0:hidden, :]
        gmz = jnp.dot(wz_ref[...], lzw1, preferred_element_type=jnp.float32)
        gmh = jnp.dot(wh_ref[...], lhw1, preferred_element_type=jnp.float32)
        gbd_ref[...] = jnp.zeros_like(gbd_ref)
        for b in range(BB):
            gbd_ref[b * f:(b + 1) * f, b * hidden:(b + 1) * hidden] = (
                gmz.astype(jnp.bfloat16))
            gbd_ref[b * f:(b + 1) * f, zw + b * hidden:zw + (b + 1) * hidden] = (
                gmh.astype(jnp.bfloat16))
        gz = jnp.dot(bz_ref[...], lzw1) + lzb_ref[...]
        gh = jnp.dot(bh_ref[...], lhw1) + lhb_ref[...]
        bias_ref[...] = jnp.concatenate([gz] * BB + [gh] * BB, axis=1)
        a = att_ref[...]
        e = jnp.exp(a - jnp.max(a))
        probs_ref[...] = e / jnp.sum(e)

    adj = adj_ref[...]
    gbd = gbd_ref[...]
    bias = bias_ref[...]
    group = 4
    acc = jnp.zeros((1, zw), jnp.float32)
    for base in range(0, periods, group):
        ps = range(base, min(base + group, periods))
        ys = [jnp.dot(adj, x_ref[p],
                      preferred_element_type=jnp.float32).astype(jnp.bfloat16)
              for p in ps]
        pres = [jnp.dot(y, gbd, preferred_element_type=jnp.float32) + bias
                for y in ys]
        for p, pre in zip(ps, pres):
            z = pre[:, 0:zw]
            h = pre[:, zw:2 * zw]
            hp = (0.5 - 0.5 * jnp.tanh(0.5 * z)) * jnp.tanh(h)
            xmean = jnp.sum(hp, axis=0, keepdims=True) * (1.0 / n)
            acc = acc + probs_ref[0, p] * xmean
    o_ref[0] = acc


def kernel(agent_obs, hideout_obs, timestep_obs, num_agents, edge_index,
           Wz, bz, Wr, br, Wh, bh, LzW, Lzb, LrW, Lrb, LhW, Lhb, attention):
    agent_obs = agent_obs.astype(jnp.float32)
    batch, periods, n, f = agent_obs.shape
    hidden = LzW.shape[1]

    xt = jnp.transpose(agent_obs.astype(jnp.bfloat16),
                       (1, 2, 0, 3)).reshape(periods, n, batch * f)

    x = pl.pallas_call(
        _body,
        grid=(batch // BB,),
        in_specs=[
            pl.BlockSpec((periods, n, BB * f), lambda i: (0, 0, i)),
            pl.BlockSpec((f, hidden), lambda i: (0, 0)),
            pl.BlockSpec((2 * hidden, hidden), lambda i: (0, 0)),
            pl.BlockSpec((1, hidden), lambda i: (0, 0)),
            pl.BlockSpec((1, hidden), lambda i: (0, 0)),
            pl.BlockSpec((f, hidden), lambda i: (0, 0)),
            pl.BlockSpec((2 * hidden, hidden), lambda i: (0, 0)),
            pl.BlockSpec((1, hidden), lambda i: (0, 0)),
            pl.BlockSpec((1, hidden), lambda i: (0, 0)),
            pl.BlockSpec((1, periods), lambda i: (0, 0)),
        ],
        out_specs=pl.BlockSpec((1, 1, BB * hidden), lambda i: (i, 0, 0)),
        out_shape=jax.ShapeDtypeStruct((batch // BB, 1, BB * hidden),
                                       jnp.float32),
        scratch_shapes=[
            pltpu.VMEM((n, n), jnp.bfloat16),
            pltpu.VMEM((BB * f, 2 * BB * hidden), jnp.bfloat16),
            pltpu.VMEM((1, 2 * BB * hidden), jnp.float32),
            pltpu.VMEM((1, periods), jnp.float32),
        ],
    )(xt, Wz, LzW, bz.reshape(1, hidden), Lzb.reshape(1, hidden),
      Wh, LhW, bh.reshape(1, hidden), Lhb.reshape(1, hidden),
      attention.reshape(1, periods))

    x = x.reshape(batch, hidden)
    return jnp.concatenate(
        [x, hideout_obs.astype(jnp.float32), timestep_obs.astype(jnp.float32)],
        axis=-1)


# speedup vs baseline: 1.1364x; 1.1364x over previous
import jax
import jax.numpy as jnp
from jax.experimental import pallas as pl
from jax.experimental.pallas import tpu as pltpu

BB = 16


def _body(x_ref, wz_ref, lzw_ref, bz_ref, lzb_ref, wh_ref, lhw_ref, bh_ref,
          lhb_ref, att_ref, o_ref, adj_ref, gbd_ref, bias_ref, probs_ref):
    i = pl.program_id(0)
    periods = x_ref.shape[0]
    n = x_ref.shape[1]
    f = wz_ref.shape[0]
    hidden = lzw_ref.shape[1]
    zw = BB * hidden

    @pl.when(i == 0)
    def _init_scratch():
        r = jax.lax.broadcasted_iota(jnp.int32, (n, n), 0)
        c = jax.lax.broadcasted_iota(jnp.int32, (n, n), 1)
        adj_ref[...] = jnp.where(
            r >= c,
            jax.lax.rsqrt(r.astype(jnp.float32) + 1.0) *
            jax.lax.rsqrt(c.astype(jnp.float32) + 1.0),
            0.0).astype(jnp.bfloat16)
        lzw1 = lzw_ref[0:hidden, :]
        lhw1 = lhw_ref[0:hidden, :]
        gmz = jnp.dot(wz_ref[...], lzw1, preferred_element_type=jnp.float32)
        gmh = jnp.dot(wh_ref[...], lhw1, preferred_element_type=jnp.float32)
        gbd_ref[...] = jnp.zeros_like(gbd_ref)
        for b in range(BB):
            gbd_ref[b * f:(b + 1) * f, b * hidden:(b + 1) * hidden] = (
                gmz.astype(jnp.bfloat16))
            gbd_ref[b * f:(b + 1) * f, zw + b * hidden:zw + (b + 1) * hidden] = (
                gmh.astype(jnp.bfloat16))
        gz = jnp.dot(bz_ref[...], lzw1) + lzb_ref[...]
        gh = jnp.dot(bh_ref[...], lhw1) + lhb_ref[...]
        bias_ref[...] = jnp.concatenate([gz] * BB + [gh] * BB, axis=1)
        a = att_ref[...]
        e = jnp.exp(a - jnp.max(a))
        probs_ref[...] = e / jnp.sum(e)

    adj = adj_ref[...]
    gbd = gbd_ref[...]
    bias = bias_ref[...]
    group = periods
    acc = jnp.zeros((1, zw), jnp.float32)
    for base in range(0, periods, group):
        ps = range(base, min(base + group, periods))
        ys = [jnp.dot(adj, x_ref[p],
                      preferred_element_type=jnp.float32).astype(jnp.bfloat16)
              for p in ps]
        pres = [jnp.dot(y, gbd, preferred_element_type=jnp.float32) + bias
                for y in ys]
        for p, pre in zip(ps, pres):
            z = pre[:, 0:zw]
            h = pre[:, zw:2 * zw]
            hp = (0.5 - 0.5 * jnp.tanh(0.5 * z)) * jnp.tanh(h)
            xmean = jnp.sum(hp, axis=0, keepdims=True) * (1.0 / n)
            acc = acc + probs_ref[0, p] * xmean
    o_ref[0] = acc


def kernel(agent_obs, hideout_obs, timestep_obs, num_agents, edge_index,
           Wz, bz, Wr, br, Wh, bh, LzW, Lzb, LrW, Lrb, LhW, Lhb, attention):
    agent_obs = agent_obs.astype(jnp.float32)
    batch, periods, n, f = agent_obs.shape
    hidden = LzW.shape[1]

    xt = jnp.transpose(agent_obs.astype(jnp.bfloat16),
                       (1, 2, 0, 3)).reshape(periods, n, batch * f)

    x = pl.pallas_call(
        _body,
        grid=(batch // BB,),
        in_specs=[
            pl.BlockSpec((periods, n, BB * f), lambda i: (0, 0, i)),
            pl.BlockSpec((f, hidden), lambda i: (0, 0)),
            pl.BlockSpec((2 * hidden, hidden), lambda i: (0, 0)),
            pl.BlockSpec((1, hidden), lambda i: (0, 0)),
            pl.BlockSpec((1, hidden), lambda i: (0, 0)),
            pl.BlockSpec((f, hidden), lambda i: (0, 0)),
            pl.BlockSpec((2 * hidden, hidden), lambda i: (0, 0)),
            pl.BlockSpec((1, hidden), lambda i: (0, 0)),
            pl.BlockSpec((1, hidden), lambda i: (0, 0)),
            pl.BlockSpec((1, periods), lambda i: (0, 0)),
        ],
        out_specs=pl.BlockSpec((1, 1, BB * hidden), lambda i: (i, 0, 0)),
        out_shape=jax.ShapeDtypeStruct((batch // BB, 1, BB * hidden),
                                       jnp.float32),
        scratch_shapes=[
            pltpu.VMEM((n, n), jnp.bfloat16),
            pltpu.VMEM((BB * f, 2 * BB * hidden), jnp.bfloat16),
            pltpu.VMEM((1, 2 * BB * hidden), jnp.float32),
            pltpu.VMEM((1, periods), jnp.float32),
        ],
    )(xt, Wz, LzW, bz.reshape(1, hidden), Lzb.reshape(1, hidden),
      Wh, LhW, bh.reshape(1, hidden), Lhb.reshape(1, hidden),
      attention.reshape(1, periods))

    x = x.reshape(batch, hidden)
    return jnp.concatenate(
        [x, hideout_obs.astype(jnp.float32), timestep_obs.astype(jnp.float32)],
        axis=-1)


# speedup vs baseline: 1.1499x; 1.0119x over previous
import jax
import jax.numpy as jnp
from jax.experimental import pallas as pl
from jax.experimental.pallas import tpu as pltpu

BB = 8


def _body(x_ref, wz_ref, lzw_ref, bz_ref, lzb_ref, wh_ref, lhw_ref, bh_ref,
          lhb_ref, att_ref, o_ref, adj_ref, gbd_ref, bias_ref, probs_ref):
    i = pl.program_id(0)
    periods = x_ref.shape[0]
    n = x_ref.shape[1]
    f = wz_ref.shape[0]
    hidden = lzw_ref.shape[1]
    zw = BB * hidden

    @pl.when(i == 0)
    def _init_scratch():
        r = jax.lax.broadcasted_iota(jnp.int32, (n, n), 0)
        c = jax.lax.broadcasted_iota(jnp.int32, (n, n), 1)
        adj_ref[...] = jnp.where(
            r >= c,
            jax.lax.rsqrt(r.astype(jnp.float32) + 1.0) *
            jax.lax.rsqrt(c.astype(jnp.float32) + 1.0),
            0.0).astype(jnp.bfloat16)
        lzw1 = lzw_ref[0:hidden, :]
        lhw1 = lhw_ref[0:hidden, :]
        gmz = jnp.dot(wz_ref[...], lzw1, preferred_element_type=jnp.float32)
        gmh = jnp.dot(wh_ref[...], lhw1, preferred_element_type=jnp.float32)
        gbd_ref[...] = jnp.zeros_like(gbd_ref)
        for b in range(BB):
            gbd_ref[b * f:(b + 1) * f, b * hidden:(b + 1) * hidden] = (
                gmz.astype(jnp.bfloat16))
            gbd_ref[b * f:(b + 1) * f, zw + b * hidden:zw + (b + 1) * hidden] = (
                gmh.astype(jnp.bfloat16))
        gz = jnp.dot(bz_ref[...], lzw1) + lzb_ref[...]
        gh = jnp.dot(bh_ref[...], lhw1) + lhb_ref[...]
        bias_ref[...] = jnp.concatenate([gz] * BB + [gh] * BB, axis=1)
        a = att_ref[...]
        e = jnp.exp(a - jnp.max(a))
        probs_ref[...] = e / jnp.sum(e)

    adj = adj_ref[...]
    gbd = gbd_ref[...]
    bias = bias_ref[...]
    group = periods
    acc = jnp.zeros((1, zw), jnp.float32)
    for base in range(0, periods, group):
        ps = range(base, min(base + group, periods))
        ys = [jnp.dot(adj, x_ref[p],
                      preferred_element_type=jnp.float32).astype(jnp.bfloat16)
              for p in ps]
        pres = [jnp.dot(y, gbd, preferred_element_type=jnp.float32) + bias
                for y in ys]
        for p, pre in zip(ps, pres):
            z = pre[:, 0:zw]
            h = pre[:, zw:2 * zw]
            hp = (0.5 - 0.5 * jnp.tanh(0.5 * z)) * jnp.tanh(h)
            xmean = jnp.sum(hp, axis=0, keepdims=True) * (1.0 / n)
            acc = acc + probs_ref[0, p] * xmean
    o_ref[0] = acc


def kernel(agent_obs, hideout_obs, timestep_obs, num_agents, edge_index,
           Wz, bz, Wr, br, Wh, bh, LzW, Lzb, LrW, Lrb, LhW, Lhb, attention):
    agent_obs = agent_obs.astype(jnp.float32)
    batch, periods, n, f = agent_obs.shape
    hidden = LzW.shape[1]

    xt = jnp.transpose(agent_obs.astype(jnp.bfloat16),
                       (1, 2, 0, 3)).reshape(periods, n, batch * f)

    x = pl.pallas_call(
        _body,
        grid=(batch // BB,),
        in_specs=[
            pl.BlockSpec((periods, n, BB * f), lambda i: (0, 0, i)),
            pl.BlockSpec((f, hidden), lambda i: (0, 0)),
            pl.BlockSpec((2 * hidden, hidden), lambda i: (0, 0)),
            pl.BlockSpec((1, hidden), lambda i: (0, 0)),
            pl.BlockSpec((1, hidden), lambda i: (0, 0)),
            pl.BlockSpec((f, hidden), lambda i: (0, 0)),
            pl.BlockSpec((2 * hidden, hidden), lambda i: (0, 0)),
            pl.BlockSpec((1, hidden), lambda i: (0, 0)),
            pl.BlockSpec((1, hidden), lambda i: (0, 0)),
            pl.BlockSpec((1, periods), lambda i: (0, 0)),
        ],
        out_specs=pl.BlockSpec((1, 1, BB * hidden), lambda i: (i, 0, 0)),
        out_shape=jax.ShapeDtypeStruct((batch // BB, 1, BB * hidden),
                                       jnp.float32),
        scratch_shapes=[
            pltpu.VMEM((n, n), jnp.bfloat16),
            pltpu.VMEM((BB * f, 2 * BB * hidden), jnp.bfloat16),
            pltpu.VMEM((1, 2 * BB * hidden), jnp.float32),
            pltpu.VMEM((1, periods), jnp.float32),
        ],
    )(xt, Wz, LzW, bz.reshape(1, hidden), Lzb.reshape(1, hidden),
      Wh, LhW, bh.reshape(1, hidden), Lhb.reshape(1, hidden),
      attention.reshape(1, periods))

    x = x.reshape(batch, hidden)
    return jnp.concatenate(
        [x, hideout_obs.astype(jnp.float32), timestep_obs.astype(jnp.float32)],
        axis=-1)


# speedup vs baseline: 1.1505x; 1.0006x over previous
import jax
import jax.numpy as jnp
from jax.experimental import pallas as pl
from jax.experimental.pallas import tpu as pltpu

BB = 8


def _body(x_ref, wz_ref, lzw_ref, bz_ref, lzb_ref, wh_ref, lhw_ref, bh_ref,
          lhb_ref, att_ref, o_ref, adj_ref, gbd_ref, bias_ref, probs_ref):
    i = pl.program_id(0)
    periods = x_ref.shape[0]
    n = x_ref.shape[1]
    f = wz_ref.shape[0]
    hidden = lzw_ref.shape[1]
    zw = BB * hidden

    @pl.when(i == 0)
    def _init_scratch():
        r = jax.lax.broadcasted_iota(jnp.int32, (n, n), 0)
        c = jax.lax.broadcasted_iota(jnp.int32, (n, n), 1)
        adj_ref[...] = jnp.where(
            r >= c,
            jax.lax.rsqrt(r.astype(jnp.float32) + 1.0) *
            jax.lax.rsqrt(c.astype(jnp.float32) + 1.0),
            0.0).astype(jnp.bfloat16)
        lzw1 = lzw_ref[0:hidden, :]
        lhw1 = lhw_ref[0:hidden, :]
        gmz = jnp.dot(wz_ref[...], lzw1, preferred_element_type=jnp.float32)
        gmh = jnp.dot(wh_ref[...], lhw1, preferred_element_type=jnp.float32)
        gbd_ref[...] = jnp.zeros_like(gbd_ref)
        for b in range(BB):
            gbd_ref[b * f:(b + 1) * f, b * hidden:(b + 1) * hidden] = (
                gmz.astype(jnp.bfloat16))
            gbd_ref[b * f:(b + 1) * f, zw + b * hidden:zw + (b + 1) * hidden] = (
                gmh.astype(jnp.bfloat16))
        gz = jnp.dot(bz_ref[...], lzw1) + lzb_ref[...]
        gh = jnp.dot(bh_ref[...], lhw1) + lhb_ref[...]
        bias_ref[...] = jnp.concatenate([gz] * BB + [gh] * BB, axis=1)
        a = att_ref[...]
        e = jnp.exp(a - jnp.max(a))
        probs_ref[...] = e / jnp.sum(e)

    adj = adj_ref[...]
    gbd = gbd_ref[...]
    bias = bias_ref[...]
    group = periods
    acc = jnp.zeros((1, zw), jnp.float32)
    for base in range(0, periods, group):
        ps = range(base, min(base + group, periods))
        ys = [jnp.dot(adj, x_ref[p],
                      preferred_element_type=jnp.float32).astype(jnp.bfloat16)
              for p in ps]
        pres = [jnp.dot(y, gbd, preferred_element_type=jnp.float32) + bias
                for y in ys]
        for p, pre in zip(ps, pres):
            z = pre[:, 0:zw]
            h = pre[:, zw:2 * zw]
            hp = (0.5 - 0.5 * jnp.tanh(0.5 * z)) * jnp.tanh(h)
            xmean = jnp.sum(hp, axis=0, keepdims=True) * (1.0 / n)
            acc = acc + probs_ref[0, p] * xmean
    o_ref[0] = acc


def kernel(agent_obs, hideout_obs, timestep_obs, num_agents, edge_index,
           Wz, bz, Wr, br, Wh, bh, LzW, Lzb, LrW, Lrb, LhW, Lhb, attention):
    agent_obs = agent_obs.astype(jnp.float32)
    batch, periods, n, f = agent_obs.shape
    hidden = LzW.shape[1]

    xt = jnp.transpose(agent_obs, (1, 2, 0, 3)).reshape(
        periods, n, batch * f).astype(jnp.bfloat16)

    x = pl.pallas_call(
        _body,
        grid=(batch // BB,),
        in_specs=[
            pl.BlockSpec((periods, n, BB * f), lambda i: (0, 0, i)),
            pl.BlockSpec((f, hidden), lambda i: (0, 0)),
            pl.BlockSpec((2 * hidden, hidden), lambda i: (0, 0)),
            pl.BlockSpec((1, hidden), lambda i: (0, 0)),
            pl.BlockSpec((1, hidden), lambda i: (0, 0)),
            pl.BlockSpec((f, hidden), lambda i: (0, 0)),
            pl.BlockSpec((2 * hidden, hidden), lambda i: (0, 0)),
            pl.BlockSpec((1, hidden), lambda i: (0, 0)),
            pl.BlockSpec((1, hidden), lambda i: (0, 0)),
            pl.BlockSpec((1, periods), lambda i: (0, 0)),
        ],
        out_specs=pl.BlockSpec((1, 1, BB * hidden), lambda i: (i, 0, 0)),
        out_shape=jax.ShapeDtypeStruct((batch // BB, 1, BB * hidden),
                                       jnp.float32),
        scratch_shapes=[
            pltpu.VMEM((n, n), jnp.bfloat16),
            pltpu.VMEM((BB * f, 2 * BB * hidden), jnp.bfloat16),
            pltpu.VMEM((1, 2 * BB * hidden), jnp.float32),
            pltpu.VMEM((1, periods), jnp.float32),
        ],
    )(xt, Wz, LzW, bz.reshape(1, hidden), Lzb.reshape(1, hidden),
      Wh, LhW, bh.reshape(1, hidden), Lhb.reshape(1, hidden),
      attention.reshape(1, periods))

    x = x.reshape(batch, hidden)
    return jnp.concatenate(
        [x, hideout_obs.astype(jnp.float32), timestep_obs.astype(jnp.float32)],
        axis=-1)
